# baseline (speedup 1.0000x reference)
import functools

import jax
import jax.numpy as jnp
from jax import lax
from jax.experimental import pallas as pl
from jax.experimental.pallas import tpu as pltpu
from jax.experimental.pallas import tpu_sc as plsc

D_MODEL = 768
MAX_REL = 127
LQ = 32
LK = 4096
NC, NS = 2, 16
HEAD = 160
GWIDE = 256
G8ROWS = 8 * GWIDE
GPER = 64
TAIL = LK - HEAD


def _sc_g_body(table_hbm, g_hbm, idx, buf, gsem):
    c = lax.axis_index("c")
    s = lax.axis_index("s")
    w = s * NC + c

    for t in range(GPER // 16):
        k16 = lax.iota(jnp.int32, 16) + (GPER * w + 16 * t)
        r16 = lax.shift_right_logical(k16, 8)
        c16 = jnp.bitwise_and(k16, GWIDE - 1)
        idx[pl.ds(16 * t, 16)] = jnp.maximum(MAX_REL + (LQ - 1) - c16 - r16, 0)
    pltpu.async_copy(table_hbm.at[idx], buf, gsem).wait()
    pltpu.sync_copy(buf, g_hbm.at[pl.ds(GPER * w, GPER)])


_sc_gather_g = functools.partial(
    pl.kernel,
    out_type=jax.ShapeDtypeStruct((G8ROWS, D_MODEL), jnp.float32),
    mesh=plsc.VectorSubcoreMesh(
        core_axis_name="c", subcore_axis_name="s", num_cores=NC, num_subcores=NS
    ),
    scratch_types=[
        pltpu.VMEM((GPER,), jnp.int32),
        pltpu.VMEM((GPER, D_MODEL), jnp.float32),
        pltpu.SemaphoreType.DMA,
    ],
)(_sc_g_body)


def _tc_bulk_body(table_ref, out_ref, const_v, sem_a, sem_b):
    const_v[...] = jnp.broadcast_to(table_ref[0:1, :], (TAIL, D_MODEL))
    h = TAIL // 2
    copies = []
    for i in range(LQ):
        base = i * LK + HEAD
        copies.append(
            pltpu.async_copy(const_v.at[pl.ds(0, h)], out_ref.at[pl.ds(base, h)], sem_a)
        )
        copies.append(
            pltpu.async_copy(const_v.at[pl.ds(h, h)], out_ref.at[pl.ds(base + h, h)], sem_b)
        )
    for cp in copies:
        cp.wait()


_tc_fill_bulk = pl.pallas_call(
    _tc_bulk_body,
    grid=(1,),
    in_specs=[pl.BlockSpec((8, D_MODEL), lambda i: (0, 0))],
    out_specs=pl.BlockSpec(memory_space=pl.ANY),
    out_shape=jax.ShapeDtypeStruct((LQ * LK, D_MODEL), jnp.float32),
    scratch_shapes=[
        pltpu.VMEM((TAIL, D_MODEL), jnp.float32),
        pltpu.SemaphoreType.DMA,
        pltpu.SemaphoreType.DMA,
    ],
)


def _tc_stitch_body(bulk_ref, g8_ref, out_ref, g8_v, lsem, sem):
    del bulk_ref
    pltpu.async_copy(g8_ref, g8_v, lsem).wait()
    copies = []
    for i in range(LQ):
        shift = (LQ - 1) - i
        r, q8 = shift % 8, (shift // 8) * 8
        src = g8_v.at[pl.ds(r * GWIDE + q8, HEAD)]
        copies.append(pltpu.async_copy(src, out_ref.at[pl.ds(i * LK, HEAD)], sem))
    for cp in copies:
        cp.wait()


_tc_stitch_head = pl.pallas_call(
    _tc_stitch_body,
    grid=(1,),
    in_specs=[
        pl.BlockSpec(memory_space=pl.ANY),
        pl.BlockSpec(memory_space=pl.ANY),
    ],
    out_specs=pl.BlockSpec(memory_space=pl.ANY),
    out_shape=jax.ShapeDtypeStruct((LQ * LK, D_MODEL), jnp.float32),
    scratch_shapes=[
        pltpu.VMEM((G8ROWS, D_MODEL), jnp.float32),
        pltpu.SemaphoreType.DMA,
        pltpu.SemaphoreType.DMA,
    ],
    input_output_aliases={0: 0},
)


def kernel(length_q, length_k, relative_embeddings):
    del length_q, length_k
    g8 = _sc_gather_g(relative_embeddings)
    bulk = _tc_fill_bulk(relative_embeddings)
    flat = _tc_stitch_head(bulk, g8)
    return flat.reshape(LQ, LK, D_MODEL)


# speedup vs baseline: 1.0349x; 1.0349x over previous
import functools

import jax
import jax.numpy as jnp
from jax import lax
from jax.experimental import pallas as pl
from jax.experimental.pallas import tpu as pltpu
from jax.experimental.pallas import tpu_sc as plsc

D_MODEL = 768
MAX_REL = 127
LQ = 32
LK = 4096
NC, NS = 2, 16
HEAD = 160
HALF = 80


def _sc_head_body(table_hbm, out_hbm, idx_a, idx_b, buf_a, buf_b, gsem, wsem):
    c = lax.axis_index("c")
    s = lax.axis_index("s")
    w = s * NC + c
    base = w * LK

    for t in range(5):
        j16 = lax.iota(jnp.int32, 16) + (16 * t)
        idx_a[pl.ds(16 * t, 16)] = jnp.maximum(MAX_REL + w - j16, 0)
    for t in range(5):
        j16 = lax.iota(jnp.int32, 16) + (16 * (t + 5))
        idx_b[pl.ds(16 * t, 16)] = jnp.maximum(MAX_REL + w - j16, 0)

    cp_a = pltpu.async_copy(table_hbm.at[idx_a], buf_a, gsem)
    cp_b = pltpu.async_copy(table_hbm.at[idx_b], buf_b, gsem)
    cp_a.wait()
    wr_a = pltpu.async_copy(buf_a, out_hbm.at[pl.ds(base, HALF)], wsem)
    cp_b.wait()
    wr_b = pltpu.async_copy(buf_b, out_hbm.at[pl.ds(base + HALF, HALF)], wsem)
    wr_a.wait()
    wr_b.wait()


_sc_gather_head = functools.partial(
    pl.kernel,
    out_type=jax.ShapeDtypeStruct((LQ * LK, D_MODEL), jnp.float32),
    mesh=plsc.VectorSubcoreMesh(
        core_axis_name="c", subcore_axis_name="s", num_cores=NC, num_subcores=NS
    ),
    scratch_types=[
        pltpu.VMEM((HALF,), jnp.int32),
        pltpu.VMEM((HALF,), jnp.int32),
        pltpu.VMEM((HALF, D_MODEL), jnp.float32),
        pltpu.VMEM((HALF, D_MODEL), jnp.float32),
        pltpu.SemaphoreType.DMA,
        pltpu.SemaphoreType.DMA,
    ],
)(_sc_head_body)


TAIL = LK - HEAD


def _tc_tail_body(partial_ref, table_ref, out_ref, const_v, sem_a, sem_b):
    del partial_ref
    const_v[...] = jnp.broadcast_to(table_ref[0:1, :], (TAIL, D_MODEL))
    h = TAIL // 2
    copies = []
    for i in range(LQ):
        base = i * LK + HEAD
        copies.append(
            pltpu.async_copy(const_v.at[pl.ds(0, h)], out_ref.at[pl.ds(base, h)], sem_a)
        )
        copies.append(
            pltpu.async_copy(const_v.at[pl.ds(h, h)], out_ref.at[pl.ds(base + h, h)], sem_b)
        )
    for cp in copies:
        cp.wait()


_tc_fill_tail = pl.pallas_call(
    _tc_tail_body,
    grid=(1,),
    in_specs=[
        pl.BlockSpec(memory_space=pl.ANY),
        pl.BlockSpec((8, D_MODEL), lambda i: (0, 0)),
    ],
    out_specs=pl.BlockSpec(memory_space=pl.ANY),
    out_shape=jax.ShapeDtypeStruct((LQ * LK, D_MODEL), jnp.float32),
    scratch_shapes=[
        pltpu.VMEM((TAIL, D_MODEL), jnp.float32),
        pltpu.SemaphoreType.DMA,
        pltpu.SemaphoreType.DMA,
    ],
    input_output_aliases={0: 0},
)


def kernel(length_q, length_k, relative_embeddings):
    del length_q, length_k
    partial = _sc_gather_head(relative_embeddings)
    flat = _tc_fill_tail(partial, relative_embeddings)
    return flat.reshape(LQ, LK, D_MODEL)


# speedup vs baseline: 1.2240x; 1.1827x over previous
import functools

import jax
import jax.numpy as jnp
from jax import lax
from jax.experimental import pallas as pl
from jax.experimental.pallas import tpu as pltpu
from jax.experimental.pallas import tpu_sc as plsc

D_MODEL = 768
MAX_REL = 127
LQ = 32
LK = 4096
NC, NS = 2, 16
HEAD = 160
GROWS = 192
TAIL = LK - HEAD


def _sc_g_body(table_hbm, g_hbm, idx, buf, gsem):
    c = lax.axis_index("c")
    s = lax.axis_index("s")
    w = s * NC + c
    base = jnp.minimum(8 * w, GROWS - 16)

    k16 = lax.iota(jnp.int32, 16) + base
    idx[...] = jnp.maximum(MAX_REL + (LQ - 1) - k16, 0)
    pltpu.async_copy(table_hbm.at[idx], buf, gsem).wait()
    pltpu.sync_copy(buf, g_hbm.at[pl.ds(base, 16)])


_sc_gather_g = functools.partial(
    pl.kernel,
    out_type=jax.ShapeDtypeStruct((GROWS, D_MODEL), jnp.float32),
    mesh=plsc.VectorSubcoreMesh(
        core_axis_name="c", subcore_axis_name="s", num_cores=NC, num_subcores=NS
    ),
    scratch_types=[
        pltpu.VMEM((16,), jnp.int32),
        pltpu.VMEM((16, D_MODEL), jnp.float32),
        pltpu.SemaphoreType.DMA,
    ],
)(_sc_g_body)


def _tc_fill_body(g_ref, table_ref, out_ref, head_v, const_v, sem_a, sem_b):
    for i in range(LQ):
        shift = (LQ - 1) - i
        head_v[pl.ds(i * HEAD, HEAD)] = g_ref[pl.ds(shift, HEAD)]
    const_v[...] = jnp.broadcast_to(table_ref[0:1, :], (TAIL, D_MODEL))
    h = TAIL // 2
    copies = []
    for i in range(LQ):
        base = i * LK
        copies.append(
            pltpu.async_copy(
                head_v.at[pl.ds(i * HEAD, HEAD)], out_ref.at[pl.ds(base, HEAD)], sem_a
            )
        )
        copies.append(
            pltpu.async_copy(
                const_v.at[pl.ds(0, h)], out_ref.at[pl.ds(base + HEAD, h)], sem_b
            )
        )
        copies.append(
            pltpu.async_copy(
                const_v.at[pl.ds(h, h)], out_ref.at[pl.ds(base + HEAD + h, h)], sem_a
            )
        )
    for cp in copies:
        cp.wait()


_tc_fill = pl.pallas_call(
    _tc_fill_body,
    grid=(1,),
    in_specs=[
        pl.BlockSpec((GROWS, D_MODEL), lambda i: (0, 0)),
        pl.BlockSpec((8, D_MODEL), lambda i: (0, 0)),
    ],
    out_specs=pl.BlockSpec(memory_space=pl.ANY),
    out_shape=jax.ShapeDtypeStruct((LQ * LK, D_MODEL), jnp.float32),
    scratch_shapes=[
        pltpu.VMEM((LQ * HEAD, D_MODEL), jnp.float32),
        pltpu.VMEM((TAIL, D_MODEL), jnp.float32),
        pltpu.SemaphoreType.DMA,
        pltpu.SemaphoreType.DMA,
    ],
)


def kernel(length_q, length_k, relative_embeddings):
    del length_q, length_k
    g = _sc_gather_g(relative_embeddings)
    flat = _tc_fill(g, relative_embeddings)
    return flat.reshape(LQ, LK, D_MODEL)


# speedup vs baseline: 1.2319x; 1.0065x over previous
import functools

import jax
import jax.numpy as jnp
from jax import lax
from jax.experimental import pallas as pl
from jax.experimental.pallas import tpu as pltpu
from jax.experimental.pallas import tpu_sc as plsc

D_MODEL = 768
MAX_REL = 127
LQ = 32
LK = 4096
NC, NS = 2, 16
HEAD = 160
GROWS = 192
TAIL = LK - HEAD


def _sc_g_body(table_hbm, g_hbm, idx, buf, gsem):
    c = lax.axis_index("c")
    s = lax.axis_index("s")
    w = s * NC + c
    base = jnp.minimum(8 * w, GROWS - 16)

    k16 = lax.iota(jnp.int32, 16) + base
    idx[...] = jnp.maximum(MAX_REL + (LQ - 1) - k16, 0)
    pltpu.async_copy(table_hbm.at[idx], buf, gsem).wait()
    pltpu.sync_copy(buf, g_hbm.at[pl.ds(base, 16)])


_sc_gather_g = functools.partial(
    pl.kernel,
    out_type=jax.ShapeDtypeStruct((GROWS, D_MODEL), jnp.float32),
    mesh=plsc.VectorSubcoreMesh(
        core_axis_name="c", subcore_axis_name="s", num_cores=NC, num_subcores=NS
    ),
    scratch_types=[
        pltpu.VMEM((16,), jnp.int32),
        pltpu.VMEM((16, D_MODEL), jnp.float32),
        pltpu.SemaphoreType.DMA,
    ],
)(_sc_g_body)


def _tc_fill_body(g_ref, table_ref, out_ref, head_v, const_v, sem_a, sem_b):
    const_v[...] = jnp.broadcast_to(table_ref[0:1, :], (TAIL, D_MODEL))
    h = TAIL // 2
    copies = []
    for i in range(LQ):
        base = i * LK + HEAD
        copies.append(
            pltpu.async_copy(const_v.at[pl.ds(0, h)], out_ref.at[pl.ds(base, h)], sem_b)
        )
        copies.append(
            pltpu.async_copy(const_v.at[pl.ds(h, h)], out_ref.at[pl.ds(base + h, h)], sem_a)
        )
    for i in range(LQ):
        shift = (LQ - 1) - i
        head_v[pl.ds(i * HEAD, HEAD)] = g_ref[pl.ds(shift, HEAD)]
    for i in range(LQ):
        copies.append(
            pltpu.async_copy(
                head_v.at[pl.ds(i * HEAD, HEAD)], out_ref.at[pl.ds(i * LK, HEAD)], sem_a
            )
        )
    for cp in copies:
        cp.wait()


_tc_fill = pl.pallas_call(
    _tc_fill_body,
    grid=(1,),
    in_specs=[
        pl.BlockSpec((GROWS, D_MODEL), lambda i: (0, 0)),
        pl.BlockSpec((8, D_MODEL), lambda i: (0, 0)),
    ],
    out_specs=pl.BlockSpec(memory_space=pl.ANY),
    out_shape=jax.ShapeDtypeStruct((LQ * LK, D_MODEL), jnp.float32),
    scratch_shapes=[
        pltpu.VMEM((LQ * HEAD, D_MODEL), jnp.float32),
        pltpu.VMEM((TAIL, D_MODEL), jnp.float32),
        pltpu.SemaphoreType.DMA,
        pltpu.SemaphoreType.DMA,
    ],
)


def kernel(length_q, length_k, relative_embeddings):
    del length_q, length_k
    g = _sc_gather_g(relative_embeddings)
    flat = _tc_fill(g, relative_embeddings)
    return flat.reshape(LQ, LK, D_MODEL)
